# baseline (speedup 1.0000x reference)
import functools

import jax
import jax.numpy as jnp
from jax.experimental import pallas as pl
from jax.experimental.pallas import tpu as pltpu

_QP = 127.0
_QN = -128.0
_EPS_CLAMP = 1e-5
_S_EPS = 2e-6


def _prep_kernel(w_ref, x_ref, ws_ref, q_ref, am_ref):
    @pl.when(pl.program_id(0) == 0)
    def _():
        ws_ref[...] = jnp.zeros_like(ws_ref)

    ws_ref[...] += jnp.sum(jnp.abs(w_ref[...]), keepdims=True)
    x = x_ref[...]
    amax = jnp.clip(jnp.max(jnp.abs(x), axis=-1, keepdims=True), _EPS_CLAMP, None)
    s_act = _QP / amax
    q_ref[...] = jnp.clip(jnp.round(x * s_act), _QN, _QP).astype(jnp.bfloat16)
    am_ref[...] = amax


def _mm_kernel(swq_ref, q_ref, w_ref, am_ref, o_ref, *, n_acc_steps, kt, mh):
    t = pl.program_id(0)
    sw = swq_ref[0, 0]

    def acc_phase(base):
        for k in range(kt):
            ks = slice(k * 256, (k + 1) * 256)
            twk = jnp.clip(
                jnp.round(w_ref[:, ks] * (sw * _QP)), -1.0, 1.0
            ).astype(jnp.bfloat16)
            r = k % 2
            pltpu.matmul_push_rhs(twk, staging_register=r, mxu_index=0,
                                  transpose=True)
            pltpu.matmul_push_rhs(twk, staging_register=r, mxu_index=1,
                                  transpose=True)
            pltpu.matmul_acc_lhs(base, q_ref[0:mh, ks], 0, load_staged_rhs=r)
            pltpu.matmul_acc_lhs(base, q_ref[mh:2 * mh, ks], 1,
                                 load_staged_rhs=r)

    def pop_phase(base):
        a0 = pltpu.matmul_pop(base, (mh, 256), jnp.float32, 0)
        a1 = pltpu.matmul_pop(base, (mh, 256), jnp.float32, 1)
        scale = (am_ref[...] + _S_EPS) * sw
        o_ref[0:mh, :] = a0 * scale[0:mh]
        o_ref[mh:2 * mh, :] = a1 * scale[mh:2 * mh]

    nb = mh // 4

    @pl.when(jnp.logical_and(t < n_acc_steps, t % 2 == 0))
    def _():
        acc_phase(0)

    @pl.when(jnp.logical_and(t < n_acc_steps, t % 2 == 1))
    def _():
        acc_phase(nb)

    @pl.when(jnp.logical_and(t > 0, t % 2 == 1))
    def _():
        pop_phase(0)

    @pl.when(jnp.logical_and(t > 0, t % 2 == 0))
    def _():
        pop_phase(nb)


def kernel(x, w):
    t_dim, k_dim = x.shape
    o_dim, _ = w.shape

    g = 1
    for cand in (32, 16, 8, 4, 2):
        if o_dim % cand == 0 and t_dim % cand == 0 \
                and (o_dim // cand) % 8 == 0 and (t_dim // cand) % 8 == 0:
            g = cand
            break
    bw = o_dim // g
    bxm = t_dim // g
    wsum, q, am = pl.pallas_call(
        _prep_kernel,
        grid=(g,),
        in_specs=[
            pl.BlockSpec((bw, k_dim), lambda i: (i, 0)),
            pl.BlockSpec((bxm, k_dim), lambda i: (i, 0)),
        ],
        out_specs=[
            pl.BlockSpec((1, 1), lambda i: (0, 0)),
            pl.BlockSpec((bxm, k_dim), lambda i: (i, 0)),
            pl.BlockSpec((bxm, 1), lambda i: (i, 0)),
        ],
        out_shape=[
            jax.ShapeDtypeStruct((1, 1), jnp.float32),
            jax.ShapeDtypeStruct((t_dim, k_dim), jnp.bfloat16),
            jax.ShapeDtypeStruct((t_dim, 1), jnp.float32),
        ],
        compiler_params=pltpu.CompilerParams(dimension_semantics=("arbitrary",)),
    )(w, x)
    s_w = 1.0 / jnp.clip(wsum / (o_dim * k_dim), _EPS_CLAMP, None)
    swq = s_w / _QP

    bm, bn = 1024, 256
    ni = t_dim // bm
    nj = o_dim // bn
    n_acc = ni * nj
    kt = k_dim // 256
    mh = bm // 2

    out = pl.pallas_call(
        functools.partial(_mm_kernel, n_acc_steps=n_acc, kt=kt, mh=mh),
        grid=(n_acc + 1,),
        in_specs=[
            pl.BlockSpec(memory_space=pltpu.SMEM),
            pl.BlockSpec((bm, k_dim),
                         lambda t: (jnp.minimum(t // nj, ni - 1), 0)),
            pl.BlockSpec((bn, k_dim), lambda t: (t % nj, 0)),
            pl.BlockSpec((bm, 1),
                         lambda t: (jnp.maximum(t - 1, 0) // nj, 0)),
        ],
        out_specs=pl.BlockSpec(
            (bm, bn),
            lambda t: (jnp.maximum(t - 1, 0) // nj, jnp.maximum(t - 1, 0) % nj),
        ),
        out_shape=jax.ShapeDtypeStruct((t_dim, o_dim), jnp.float32),
        compiler_params=pltpu.CompilerParams(
            dimension_semantics=("arbitrary",),
            vmem_limit_bytes=58 * 1024 * 1024,
        ),
    )(swq, q, w, am)
    return out


# speedup vs baseline: 1.1424x; 1.1424x over previous
import functools

import jax
import jax.numpy as jnp
from jax.experimental import pallas as pl
from jax.experimental.pallas import tpu as pltpu

_QP = 127.0
_QN = -128.0
_EPS_CLAMP = 1e-5
_S_EPS = 2e-6


def _prep_kernel(w_ref, x_ref, ws_ref, q_ref, am_ref):
    @pl.when(pl.program_id(0) == 0)
    def _():
        ws_ref[...] = jnp.zeros_like(ws_ref)

    ws_ref[...] += jnp.sum(jnp.abs(w_ref[...]), keepdims=True)
    x = x_ref[...]
    amax = jnp.clip(jnp.max(jnp.abs(x), axis=-1, keepdims=True), _EPS_CLAMP, None)
    s_act = _QP / amax
    q_ref[...] = jnp.clip(jnp.round(x * s_act), _QN, _QP).astype(jnp.bfloat16)
    am_ref[...] = amax


def _mm_kernel(swq_ref, q_ref, w_ref, am_ref, o_ref, *, n_acc_steps, kt, mh):
    t = pl.program_id(0)
    sw = swq_ref[0, 0]

    a0 = pltpu.matmul_pop(0, (mh, 256), jnp.float32, 0)
    a1 = pltpu.matmul_pop(0, (mh, 256), jnp.float32, 1)
    scale = (am_ref[...] + _S_EPS) * sw
    o_ref[0:mh, :] = a0 * scale[0:mh]
    o_ref[mh:2 * mh, :] = a1 * scale[mh:2 * mh]

    for k in range(kt):
        ks = slice(k * 256, (k + 1) * 256)
        twk = jnp.clip(
            jnp.round(w_ref[:, ks] * (sw * _QP)), -1.0, 1.0
        ).astype(jnp.bfloat16)
        r = k % 2
        pltpu.matmul_push_rhs(twk, staging_register=r, mxu_index=0,
                              transpose=True)
        pltpu.matmul_push_rhs(twk, staging_register=r, mxu_index=1,
                              transpose=True)
        pltpu.matmul_acc_lhs(0, q_ref[0:mh, ks], 0, load_staged_rhs=r)
        pltpu.matmul_acc_lhs(0, q_ref[mh:2 * mh, ks], 1, load_staged_rhs=r)

    @pl.when(t == n_acc_steps)
    def _():
        pltpu.matmul_pop(0, (mh, 256), jnp.float32, 0)
        pltpu.matmul_pop(0, (mh, 256), jnp.float32, 1)


def kernel(x, w):
    t_dim, k_dim = x.shape
    o_dim, _ = w.shape

    g = 1
    for cand in (32, 16, 8, 4, 2):
        if o_dim % cand == 0 and t_dim % cand == 0 \
                and (o_dim // cand) % 8 == 0 and (t_dim // cand) % 8 == 0:
            g = cand
            break
    bw = o_dim // g
    bxm = t_dim // g
    wsum, q, am = pl.pallas_call(
        _prep_kernel,
        grid=(g,),
        in_specs=[
            pl.BlockSpec((bw, k_dim), lambda i: (i, 0)),
            pl.BlockSpec((bxm, k_dim), lambda i: (i, 0)),
        ],
        out_specs=[
            pl.BlockSpec((1, 1), lambda i: (0, 0)),
            pl.BlockSpec((bxm, k_dim), lambda i: (i, 0)),
            pl.BlockSpec((bxm, 1), lambda i: (i, 0)),
        ],
        out_shape=[
            jax.ShapeDtypeStruct((1, 1), jnp.float32),
            jax.ShapeDtypeStruct((t_dim, k_dim), jnp.bfloat16),
            jax.ShapeDtypeStruct((t_dim, 1), jnp.float32),
        ],
        compiler_params=pltpu.CompilerParams(dimension_semantics=("arbitrary",)),
    )(w, x)
    s_w = 1.0 / jnp.clip(wsum / (o_dim * k_dim), _EPS_CLAMP, None)
    swq = s_w / _QP

    bm, bn = 2048, 256
    ni = t_dim // bm
    nj = o_dim // bn
    n_acc = ni * nj
    kt = k_dim // 256
    mh = bm // 2

    out = pl.pallas_call(
        functools.partial(_mm_kernel, n_acc_steps=n_acc, kt=kt, mh=mh),
        grid=(n_acc + 1,),
        in_specs=[
            pl.BlockSpec(memory_space=pltpu.SMEM),
            pl.BlockSpec((bm, k_dim),
                         lambda t: (jnp.minimum(t // nj, ni - 1), 0)),
            pl.BlockSpec((bn, k_dim), lambda t: (t % nj, 0)),
            pl.BlockSpec((bm, 1),
                         lambda t: (jnp.maximum(t - 1, 0) // nj, 0)),
        ],
        out_specs=pl.BlockSpec(
            (bm, bn),
            lambda t: (jnp.maximum(t - 1, 0) // nj, jnp.maximum(t - 1, 0) % nj),
        ),
        out_shape=jax.ShapeDtypeStruct((t_dim, o_dim), jnp.float32),
        compiler_params=pltpu.CompilerParams(
            dimension_semantics=("arbitrary",),
            vmem_limit_bytes=58 * 1024 * 1024,
        ),
    )(swq, q, w, am)
    return out


# speedup vs baseline: 1.1435x; 1.0009x over previous
import functools

import jax
import jax.numpy as jnp
from jax.experimental import pallas as pl
from jax.experimental.pallas import tpu as pltpu

_QP = 127.0
_QN = -128.0
_EPS_CLAMP = 1e-5
_S_EPS = 2e-6


def _prep_kernel(w_ref, x_ref, ws_ref, q_ref, am_ref):
    @pl.when(pl.program_id(0) == 0)
    def _():
        ws_ref[...] = jnp.zeros_like(ws_ref)

    ws_ref[...] += jnp.sum(jnp.abs(w_ref[...]), keepdims=True)
    x = x_ref[...]
    amax = jnp.clip(jnp.max(jnp.abs(x), axis=-1, keepdims=True), _EPS_CLAMP, None)
    s_act = _QP / amax
    q_ref[...] = jnp.clip(jnp.round(x * s_act), _QN, _QP).astype(jnp.bfloat16)
    am_ref[...] = amax


def _mm_kernel(swq_ref, q_ref, w_ref, am_ref, o_ref, *, n_acc_steps, kt, mh):
    t = pl.program_id(0)
    sw = swq_ref[0, 0]

    def quant_push(k):
        ks = slice(k * 256, (k + 1) * 256)
        twk = jnp.clip(
            jnp.round(w_ref[:, ks] * (sw * _QP)), -1.0, 1.0
        ).astype(jnp.bfloat16)
        r = k % 2
        pltpu.matmul_push_rhs(twk, staging_register=r, mxu_index=0,
                              transpose=True)
        pltpu.matmul_push_rhs(twk, staging_register=r, mxu_index=1,
                              transpose=True)

    def acc(k):
        ks = slice(k * 256, (k + 1) * 256)
        r = k % 2
        pltpu.matmul_acc_lhs(0, q_ref[0:mh, ks], 0, load_staged_rhs=r)
        pltpu.matmul_acc_lhs(0, q_ref[mh:2 * mh, ks], 1, load_staged_rhs=r)

    quant_push(0)

    a0 = pltpu.matmul_pop(0, (mh, 256), jnp.float32, 0)
    a1 = pltpu.matmul_pop(0, (mh, 256), jnp.float32, 1)
    scale = (am_ref[...] + _S_EPS) * sw
    o_ref[0:mh, :] = a0 * scale[0:mh]
    o_ref[mh:2 * mh, :] = a1 * scale[mh:2 * mh]

    for k in range(kt):
        if k > 0:
            quant_push(k)
        acc(k)

    @pl.when(t == n_acc_steps)
    def _():
        pltpu.matmul_pop(0, (mh, 256), jnp.float32, 0)
        pltpu.matmul_pop(0, (mh, 256), jnp.float32, 1)


def kernel(x, w):
    t_dim, k_dim = x.shape
    o_dim, _ = w.shape

    g = 1
    for cand in (32, 16, 8, 4, 2):
        if o_dim % cand == 0 and t_dim % cand == 0 \
                and (o_dim // cand) % 8 == 0 and (t_dim // cand) % 8 == 0:
            g = cand
            break
    bw = o_dim // g
    bxm = t_dim // g
    wsum, q, am = pl.pallas_call(
        _prep_kernel,
        grid=(g,),
        in_specs=[
            pl.BlockSpec((bw, k_dim), lambda i: (i, 0)),
            pl.BlockSpec((bxm, k_dim), lambda i: (i, 0)),
        ],
        out_specs=[
            pl.BlockSpec((1, 1), lambda i: (0, 0)),
            pl.BlockSpec((bxm, k_dim), lambda i: (i, 0)),
            pl.BlockSpec((bxm, 1), lambda i: (i, 0)),
        ],
        out_shape=[
            jax.ShapeDtypeStruct((1, 1), jnp.float32),
            jax.ShapeDtypeStruct((t_dim, k_dim), jnp.bfloat16),
            jax.ShapeDtypeStruct((t_dim, 1), jnp.float32),
        ],
        compiler_params=pltpu.CompilerParams(dimension_semantics=("arbitrary",)),
    )(w, x)
    s_w = 1.0 / jnp.clip(wsum / (o_dim * k_dim), _EPS_CLAMP, None)
    swq = s_w / _QP

    bm, bn = 2048, 256
    ni = t_dim // bm
    nj = o_dim // bn
    n_acc = ni * nj
    kt = k_dim // 256
    mh = bm // 2

    out = pl.pallas_call(
        functools.partial(_mm_kernel, n_acc_steps=n_acc, kt=kt, mh=mh),
        grid=(n_acc + 1,),
        in_specs=[
            pl.BlockSpec(memory_space=pltpu.SMEM),
            pl.BlockSpec((bm, k_dim),
                         lambda t: (jnp.minimum(t // nj, ni - 1), 0)),
            pl.BlockSpec((bn, k_dim), lambda t: (t % nj, 0)),
            pl.BlockSpec((bm, 1),
                         lambda t: (jnp.maximum(t - 1, 0) // nj, 0)),
        ],
        out_specs=pl.BlockSpec(
            (bm, bn),
            lambda t: (jnp.maximum(t - 1, 0) // nj, jnp.maximum(t - 1, 0) % nj),
        ),
        out_shape=jax.ShapeDtypeStruct((t_dim, o_dim), jnp.float32),
        compiler_params=pltpu.CompilerParams(
            dimension_semantics=("arbitrary",),
            vmem_limit_bytes=58 * 1024 * 1024,
        ),
    )(swq, q, w, am)
    return out
